# speedup vs baseline: 8.7574x; 8.7574x over previous
import functools

import jax
import jax.numpy as jnp
from jax import lax
from jax.experimental import pallas as pl
from jax.experimental.pallas import tpu as pltpu
from jax.experimental.pallas import tpu_sc as plsc

NC = 2
NS = 16
NW = NC * NS
D = 128
N = 128


@functools.partial(jax.jit, static_argnames=("bc",))
def _run(src, pth, tgt, token_table, path_table, bc):
    per_w = bc // NW
    steps = per_w // N
    mesh = plsc.VectorSubcoreMesh(core_axis_name="c", subcore_axis_name="s")

    @functools.partial(
        pl.kernel,
        mesh=mesh,
        out_type=jax.ShapeDtypeStruct((bc, 3 * D), jnp.float32),
        scratch_types=[
            pltpu.VMEM((per_w,), jnp.int32),
            pltpu.VMEM((per_w,), jnp.int32),
            pltpu.VMEM((per_w,), jnp.int32),
            pltpu.VMEM((N, D), jnp.float32),
            pltpu.VMEM((N, D), jnp.float32),
            pltpu.VMEM((N, D), jnp.float32),
            pltpu.SemaphoreType.DMA,
        ],
    )
    def k(src_hbm, pth_hbm, tgt_hbm, token_hbm, path_hbm, out_hbm,
          src_v, pth_v, tgt_v, buf_s, buf_p, buf_t, sem):
        wid = lax.axis_index("s") * NC + lax.axis_index("c")
        base = wid * per_w
        pltpu.sync_copy(src_hbm.at[pl.ds(base, per_w)], src_v)
        pltpu.sync_copy(pth_hbm.at[pl.ds(base, per_w)], pth_v)
        pltpu.sync_copy(tgt_hbm.at[pl.ds(base, per_w)], tgt_v)

        def step(i, carry):
            off = i * N
            row0 = base + off
            cs = pltpu.async_copy(token_hbm.at[src_v.at[pl.ds(off, N)]], buf_s, sem)
            cp = pltpu.async_copy(path_hbm.at[pth_v.at[pl.ds(off, N)]], buf_p, sem)
            ct = pltpu.async_copy(token_hbm.at[tgt_v.at[pl.ds(off, N)]], buf_t, sem)
            cs.wait()
            cp.wait()
            ct.wait()
            pltpu.sync_copy(buf_s, out_hbm.at[pl.ds(row0, N), pl.ds(0, D)])
            pltpu.sync_copy(buf_p, out_hbm.at[pl.ds(row0, N), pl.ds(D, D)])
            pltpu.sync_copy(buf_t, out_hbm.at[pl.ds(row0, N), pl.ds(2 * D, D)])
            return carry

        lax.fori_loop(0, steps, step, 0)

    return k(src, pth, tgt, token_table, path_table)


def kernel(path_source_token_idxs, path_idxs, path_target_token_idxs, token_table, path_table):
    b, c = path_source_token_idxs.shape
    bc = b * c
    src = path_source_token_idxs.reshape(bc).astype(jnp.int32)
    pth = path_idxs.reshape(bc).astype(jnp.int32)
    tgt = path_target_token_idxs.reshape(bc).astype(jnp.int32)
    out = _run(src, pth, tgt, token_table, path_table, bc)
    return out.reshape(b, c, 3 * D)


# speedup vs baseline: 9.6677x; 1.1039x over previous
import functools

import jax
import jax.numpy as jnp
from jax import lax
from jax.experimental import pallas as pl
from jax.experimental.pallas import tpu as pltpu
from jax.experimental.pallas import tpu_sc as plsc

NC = 2
NS = 16
NW = NC * NS
D = 128
N = 128


@functools.partial(jax.jit, static_argnames=("bc",))
def _run(src, pth, tgt, token_table, path_table, bc):
    per_w = bc // NW
    steps = per_w // N
    pairs = steps // 2
    mesh = plsc.VectorSubcoreMesh(core_axis_name="c", subcore_axis_name="s")

    @functools.partial(
        pl.kernel,
        mesh=mesh,
        out_type=jax.ShapeDtypeStruct((bc, 3 * D), jnp.float32),
        scratch_types=[
            pltpu.VMEM((per_w,), jnp.int32),
            pltpu.VMEM((per_w,), jnp.int32),
            pltpu.VMEM((per_w,), jnp.int32),
            pltpu.VMEM((N, D), jnp.float32),
            pltpu.VMEM((N, D), jnp.float32),
            pltpu.VMEM((N, D), jnp.float32),
            pltpu.VMEM((N, D), jnp.float32),
            pltpu.VMEM((N, D), jnp.float32),
            pltpu.VMEM((N, D), jnp.float32),
            pltpu.SemaphoreType.DMA,
            pltpu.SemaphoreType.DMA,
            pltpu.SemaphoreType.DMA,
            pltpu.SemaphoreType.DMA,
        ],
    )
    def k(src_hbm, pth_hbm, tgt_hbm, token_hbm, path_hbm, out_hbm,
          src_v, pth_v, tgt_v, b0s, b0p, b0t, b1s, b1p, b1t,
          g0, g1, w0, w1):
        wid = lax.axis_index("s") * NC + lax.axis_index("c")
        base = wid * per_w
        pltpu.sync_copy(src_hbm.at[pl.ds(base, per_w)], src_v)
        pltpu.sync_copy(pth_hbm.at[pl.ds(base, per_w)], pth_v)
        pltpu.sync_copy(tgt_hbm.at[pl.ds(base, per_w)], tgt_v)

        sets = ((b0s, b0p, b0t, g0, w0), (b1s, b1p, b1t, g1, w1))

        def gfire(t, s):
            bs, bp, bt, g, _ = sets[s]
            off = t * N
            pltpu.async_copy(token_hbm.at[src_v.at[pl.ds(off, N)]], bs, g)
            pltpu.async_copy(path_hbm.at[pth_v.at[pl.ds(off, N)]], bp, g)
            pltpu.async_copy(token_hbm.at[tgt_v.at[pl.ds(off, N)]], bt, g)

        def gwait(s):
            bs, bp, bt, g, _ = sets[s]
            for b in (bs, bp, bt):
                pltpu.make_async_copy(out_hbm.at[pl.ds(0, N), pl.ds(0, D)], b, g).wait()

        def wfire(t, s):
            bs, bp, bt, _, w = sets[s]
            row0 = base + t * N
            pltpu.async_copy(bs, out_hbm.at[pl.ds(row0, N), pl.ds(0, D)], w)
            pltpu.async_copy(bp, out_hbm.at[pl.ds(row0, N), pl.ds(D, D)], w)
            pltpu.async_copy(bt, out_hbm.at[pl.ds(row0, N), pl.ds(2 * D, D)], w)

        def wwait(s):
            bs, bp, bt, _, w = sets[s]
            for b in (bs, bp, bt):
                pltpu.make_async_copy(b, out_hbm.at[pl.ds(0, N), pl.ds(0, D)], w).wait()

        gfire(0, 0)

        def body(i, carry):
            t0 = 2 * i
            gwait(0)
            wfire(t0, 0)
            with jax.named_scope("drain_w1"):
                @pl.when(i > 0)
                def _():
                    wwait(1)
            gfire(t0 + 1, 1)
            gwait(1)
            wfire(t0 + 1, 1)
            wwait(0)
            with jax.named_scope("next_even"):
                @pl.when(i < pairs - 1)
                def _():
                    gfire(t0 + 2, 0)
            return carry

        lax.fori_loop(0, pairs, body, 0)
        wwait(1)

    return k(src, pth, tgt, token_table, path_table)


def kernel(path_source_token_idxs, path_idxs, path_target_token_idxs, token_table, path_table):
    b, c = path_source_token_idxs.shape
    bc = b * c
    src = path_source_token_idxs.reshape(bc).astype(jnp.int32)
    pth = path_idxs.reshape(bc).astype(jnp.int32)
    tgt = path_target_token_idxs.reshape(bc).astype(jnp.int32)
    out = _run(src, pth, tgt, token_table, path_table, bc)
    return out.reshape(b, c, 3 * D)


# speedup vs baseline: 9.6758x; 1.0008x over previous
import functools

import jax
import jax.numpy as jnp
from jax import lax
from jax.experimental import pallas as pl
from jax.experimental.pallas import tpu as pltpu
from jax.experimental.pallas import tpu_sc as plsc

NC = 2
NS = 16
NW = NC * NS
D = 128
N = 128


@functools.partial(jax.jit, static_argnames=("bc",))
def _run(src, pth, tgt, token_table, path_table, bc):
    per_w = bc // NW
    steps = per_w // N
    pairs = steps // 2
    mesh = plsc.VectorSubcoreMesh(core_axis_name="c", subcore_axis_name="s")

    @functools.partial(
        pl.kernel,
        mesh=mesh,
        out_type=jax.ShapeDtypeStruct((bc, 3 * D), jnp.float32),
        scratch_types=[
            pltpu.VMEM((per_w,), jnp.int32),
            pltpu.VMEM((per_w,), jnp.int32),
            pltpu.VMEM((per_w,), jnp.int32),
            pltpu.VMEM((N, 3 * D), jnp.float32),
            pltpu.VMEM((N, 3 * D), jnp.float32),
            pltpu.SemaphoreType.DMA,
            pltpu.SemaphoreType.DMA,
            pltpu.SemaphoreType.DMA,
            pltpu.SemaphoreType.DMA,
        ],
    )
    def k(src_hbm, pth_hbm, tgt_hbm, token_hbm, path_hbm, out_hbm,
          src_v, pth_v, tgt_v, b0, b1,
          g0, g1, w0, w1):
        wid = lax.axis_index("s") * NC + lax.axis_index("c")
        base = wid * per_w
        pltpu.sync_copy(src_hbm.at[pl.ds(base, per_w)], src_v)
        pltpu.sync_copy(pth_hbm.at[pl.ds(base, per_w)], pth_v)
        pltpu.sync_copy(tgt_hbm.at[pl.ds(base, per_w)], tgt_v)

        sets = ((b0, g0, w0), (b1, g1, w1))

        def gfire(t, s):
            b, g, _ = sets[s]
            off = t * N
            pltpu.async_copy(token_hbm.at[src_v.at[pl.ds(off, N)]], b.at[:, pl.ds(0, D)], g)
            pltpu.async_copy(path_hbm.at[pth_v.at[pl.ds(off, N)]], b.at[:, pl.ds(D, D)], g)
            pltpu.async_copy(token_hbm.at[tgt_v.at[pl.ds(off, N)]], b.at[:, pl.ds(2 * D, D)], g)

        def gwait(s):
            b, g, _ = sets[s]
            for f in range(3):
                pltpu.make_async_copy(out_hbm.at[pl.ds(0, N), pl.ds(0, D)], b.at[:, pl.ds(f * D, D)], g).wait()

        def wfire(t, s):
            b, _, w = sets[s]
            row0 = base + t * N
            pltpu.async_copy(b, out_hbm.at[pl.ds(row0, N)], w)

        def wwait(s):
            b, _, w = sets[s]
            pltpu.make_async_copy(b, out_hbm.at[pl.ds(0, N)], w).wait()

        gfire(0, 0)

        def body(i, carry):
            t0 = 2 * i
            gwait(0)
            wfire(t0, 0)
            with jax.named_scope("drain_w1"):
                @pl.when(i > 0)
                def _():
                    wwait(1)
            gfire(t0 + 1, 1)
            gwait(1)
            wfire(t0 + 1, 1)
            wwait(0)
            with jax.named_scope("next_even"):
                @pl.when(i < pairs - 1)
                def _():
                    gfire(t0 + 2, 0)
            return carry

        lax.fori_loop(0, pairs, body, 0)
        wwait(1)

    return k(src, pth, tgt, token_table, path_table)


def kernel(path_source_token_idxs, path_idxs, path_target_token_idxs, token_table, path_table):
    b, c = path_source_token_idxs.shape
    bc = b * c
    src = path_source_token_idxs.reshape(bc).astype(jnp.int32)
    pth = path_idxs.reshape(bc).astype(jnp.int32)
    tgt = path_target_token_idxs.reshape(bc).astype(jnp.int32)
    out = _run(src, pth, tgt, token_table, path_table, bc)
    return out.reshape(b, c, 3 * D)


# speedup vs baseline: 9.7427x; 1.0069x over previous
import functools

import jax
import jax.numpy as jnp
from jax import lax
from jax.experimental import pallas as pl
from jax.experimental.pallas import tpu as pltpu
from jax.experimental.pallas import tpu_sc as plsc

NC = 2
NS = 16
NW = NC * NS
D = 128
N = 64
S = 4


@functools.partial(jax.jit, static_argnames=("bc",))
def _run(src, pth, tgt, token_table, path_table, bc):
    per_w = bc // NW
    steps = per_w // N
    groups = steps // S
    mesh = plsc.VectorSubcoreMesh(core_axis_name="c", subcore_axis_name="s")

    @functools.partial(
        pl.kernel,
        mesh=mesh,
        out_type=jax.ShapeDtypeStruct((bc, 3 * D), jnp.float32),
        scratch_types=[
            pltpu.VMEM((per_w,), jnp.int32),
            pltpu.VMEM((per_w,), jnp.int32),
            pltpu.VMEM((per_w,), jnp.int32),
        ] + [pltpu.VMEM((N, 3 * D), jnp.float32)] * S
          + [pltpu.SemaphoreType.DMA] * (2 * S),
    )
    def k(src_hbm, pth_hbm, tgt_hbm, token_hbm, path_hbm, out_hbm,
          src_v, pth_v, tgt_v, *bufs_and_sems):
        bufs = bufs_and_sems[:S]
        gsems = bufs_and_sems[S:2 * S]
        wsems = bufs_and_sems[2 * S:3 * S]
        wid = lax.axis_index("s") * NC + lax.axis_index("c")
        base = wid * per_w
        pltpu.sync_copy(src_hbm.at[pl.ds(base, per_w)], src_v)
        pltpu.sync_copy(pth_hbm.at[pl.ds(base, per_w)], pth_v)
        pltpu.sync_copy(tgt_hbm.at[pl.ds(base, per_w)], tgt_v)

        sets = tuple((bufs[s], gsems[s], wsems[s]) for s in range(S))

        def gfire(t, s):
            b, g, _ = sets[s]
            off = t * N
            pltpu.async_copy(token_hbm.at[src_v.at[pl.ds(off, N)]], b.at[:, pl.ds(0, D)], g)
            pltpu.async_copy(path_hbm.at[pth_v.at[pl.ds(off, N)]], b.at[:, pl.ds(D, D)], g)
            pltpu.async_copy(token_hbm.at[tgt_v.at[pl.ds(off, N)]], b.at[:, pl.ds(2 * D, D)], g)

        def gwait(s):
            b, g, _ = sets[s]
            for f in range(3):
                pltpu.make_async_copy(out_hbm.at[pl.ds(0, N), pl.ds(0, D)], b.at[:, pl.ds(f * D, D)], g).wait()

        def wfire(t, s):
            b, _, w = sets[s]
            row0 = base + t * N
            pltpu.async_copy(b, out_hbm.at[pl.ds(row0, N)], w)

        def wwait(s):
            b, _, w = sets[s]
            pltpu.make_async_copy(b, out_hbm.at[pl.ds(0, N)], w).wait()

        gfire(0, 0)
        gfire(1, 1)

        def body(i, carry):
            for a in range(S):
                t = S * i + a
                gwait(a)
                wfire(t, a)
                sn = (a + 2) % S
                with jax.named_scope("drain_w"):
                    @pl.when(t >= 2)
                    def _():
                        wwait(sn)
                with jax.named_scope("next_g"):
                    @pl.when(t + 2 < steps)
                    def _():
                        gfire(t + 2, sn)
            return carry

        lax.fori_loop(0, groups, body, 0)
        wwait((steps - 2) % S)
        wwait((steps - 1) % S)

    return k(src, pth, tgt, token_table, path_table)


def kernel(path_source_token_idxs, path_idxs, path_target_token_idxs, token_table, path_table):
    b, c = path_source_token_idxs.shape
    bc = b * c
    src = path_source_token_idxs.reshape(bc).astype(jnp.int32)
    pth = path_idxs.reshape(bc).astype(jnp.int32)
    tgt = path_target_token_idxs.reshape(bc).astype(jnp.int32)
    out = _run(src, pth, tgt, token_table, path_table, bc)
    return out.reshape(b, c, 3 * D)


# speedup vs baseline: 9.8253x; 1.0085x over previous
import functools

import jax
import jax.numpy as jnp
from jax import lax
from jax.experimental import pallas as pl
from jax.experimental.pallas import tpu as pltpu
from jax.experimental.pallas import tpu_sc as plsc

NC = 2
NS = 16
NW = NC * NS
D = 128
N = 64
S = 4


@functools.partial(jax.jit, static_argnames=("bc",))
def _run(src, pth, tgt, token_table, path_table, bc):
    per_w = bc // NW
    steps = per_w // N
    groups = steps // S
    mesh = plsc.VectorSubcoreMesh(core_axis_name="c", subcore_axis_name="s")

    @functools.partial(
        pl.kernel,
        mesh=mesh,
        out_type=jax.ShapeDtypeStruct((bc, 3 * D), jnp.float32),
        scratch_types=[
            pltpu.VMEM((per_w,), jnp.int32),
            pltpu.VMEM((per_w,), jnp.int32),
            pltpu.VMEM((per_w,), jnp.int32),
        ] + [pltpu.VMEM((N, 3 * D), jnp.float32)] * S
          + [pltpu.SemaphoreType.DMA] * (4 * S),
    )
    def k(src_hbm, pth_hbm, tgt_hbm, token_hbm, path_hbm, out_hbm,
          src_v, pth_v, tgt_v, *bufs_and_sems):
        bufs = bufs_and_sems[:S]
        gsems = [bufs_and_sems[S + 3 * s:S + 3 * s + 3] for s in range(S)]
        wsems = bufs_and_sems[4 * S:5 * S]
        wid = lax.axis_index("s") * NC + lax.axis_index("c")
        base = wid * per_w
        pltpu.sync_copy(src_hbm.at[pl.ds(base, per_w)], src_v)
        pltpu.sync_copy(pth_hbm.at[pl.ds(base, per_w)], pth_v)
        pltpu.sync_copy(tgt_hbm.at[pl.ds(base, per_w)], tgt_v)

        sets = tuple((bufs[s], gsems[s], wsems[s]) for s in range(S))

        def gfire(t, s):
            b, g, _ = sets[s]
            off = t * N
            pltpu.async_copy(token_hbm.at[src_v.at[pl.ds(off, N)]], b.at[:, pl.ds(0, D)], g[0])
            pltpu.async_copy(path_hbm.at[pth_v.at[pl.ds(off, N)]], b.at[:, pl.ds(D, D)], g[1])
            pltpu.async_copy(token_hbm.at[tgt_v.at[pl.ds(off, N)]], b.at[:, pl.ds(2 * D, D)], g[2])

        def gwait_band(s, f):
            b, g, _ = sets[s]
            pltpu.make_async_copy(out_hbm.at[pl.ds(0, N), pl.ds(0, D)], b.at[:, pl.ds(f * D, D)], g[f]).wait()

        def wfire_band(t, s, f):
            b, _, w = sets[s]
            row0 = base + t * N
            pltpu.async_copy(b.at[:, pl.ds(f * D, D)], out_hbm.at[pl.ds(row0, N), pl.ds(f * D, D)], w)

        def wwait(s):
            b, _, w = sets[s]
            for f in range(3):
                pltpu.make_async_copy(b.at[:, pl.ds(f * D, D)], out_hbm.at[pl.ds(0, N), pl.ds(0, D)], w).wait()

        gfire(0, 0)
        gfire(1, 1)

        def body(i, carry):
            for a in range(S):
                t = S * i + a
                for f in range(3):
                    gwait_band(a, f)
                    wfire_band(t, a, f)
                sn = (a + 2) % S
                with jax.named_scope("drain_w"):
                    @pl.when(t >= 2)
                    def _():
                        wwait(sn)
                with jax.named_scope("next_g"):
                    @pl.when(t + 2 < steps)
                    def _():
                        gfire(t + 2, sn)
            return carry

        lax.fori_loop(0, groups, body, 0)
        wwait((steps - 2) % S)
        wwait((steps - 1) % S)

    return k(src, pth, tgt, token_table, path_table)


def kernel(path_source_token_idxs, path_idxs, path_target_token_idxs, token_table, path_table):
    b, c = path_source_token_idxs.shape
    bc = b * c
    src = path_source_token_idxs.reshape(bc).astype(jnp.int32)
    pth = path_idxs.reshape(bc).astype(jnp.int32)
    tgt = path_target_token_idxs.reshape(bc).astype(jnp.int32)
    out = _run(src, pth, tgt, token_table, path_table, bc)
    return out.reshape(b, c, 3 * D)
